# baseline (speedup 1.0000x reference)
import functools

import jax
import jax.numpy as jnp
from jax import lax
from jax.experimental import pallas as pl
from jax.experimental.pallas import tpu as pltpu
from jax.experimental.pallas import tpu_sc as plsc

N = 10000
E = 320000
D = 128
DE = 16

NC = 2
NS = 16
NW = NC * NS
EPW = E // NW
CK = 80
NCHUNK = EPW // CK
RPT = N // 10

_mesh = plsc.VectorSubcoreMesh(core_axis_name="c", subcore_axis_name="s")


def _mm_body(x_ref, w_ref, b_ref, o_ref):
    o_ref[...] = jnp.dot(x_ref[...], w_ref[...],
                         preferred_element_type=jnp.float32) + b_ref[...]


def _mm(x, w, b, block_rows):
    n, d_in = x.shape
    d_out = w.shape[1]
    return pl.pallas_call(
        _mm_body,
        grid=(n // block_rows,),
        in_specs=[
            pl.BlockSpec((block_rows, d_in), lambda i: (i, 0)),
            pl.BlockSpec((d_in, d_out), lambda i: (0, 0)),
            pl.BlockSpec((d_out,), lambda i: (0,)),
        ],
        out_specs=pl.BlockSpec((block_rows, d_out), lambda i: (i, 0)),
        out_shape=jax.ShapeDtypeStruct((n, d_out), jnp.float32),
    )(x, w, b)


def _k0_body(dst_hbm, ew_hbm,
             deg_out, ews_out,
             ews_s, deg_s,
             dst0, dst1, sd0, sd1, ew0, ew1, one_v, z_v, z1_v,
             sdst0, sdst1, sew0, sew1, sews0, sews1, sone0, sone1):
    cid = lax.axis_index("c")
    sid = lax.axis_index("s")
    wid = cid * NS + sid

    DST = (dst0, dst1)
    SD = (sd0, sd1)
    EW = (ew0, ew1)
    SDST = (sdst0, sdst1)
    SEW = (sew0, sew1)
    SEWS = (sews0, sews1)
    SONE = (sone0, sone1)

    def _zrow(i, _):
        for c in range(8):
            z_v[i, pl.ds(c * 16, 16)] = jnp.zeros((16,), jnp.float32)
        return 0
    lax.fori_loop(0, 100, _zrow, 0)

    def _z1(i, _):
        z1_v[pl.ds(i * 16, 16)] = jnp.zeros((16,), jnp.float32)
        return 0
    lax.fori_loop(0, 63, _z1, 0)

    def _ones(i, _):
        one_v[pl.ds(i * 16, 16)] = jnp.ones((16,), jnp.float32)
        return 0
    lax.fori_loop(0, CK // 16, _ones, 0)

    @pl.when(sid < 10)
    def _init():
        base = sid * RPT
        pltpu.sync_copy(z1_v.at[pl.ds(0, RPT)], deg_s.at[pl.ds(base, RPT)])
        def _zi(j, _):
            pltpu.sync_copy(z_v, ews_s.at[pl.ds(base + j * 100, 100), :])
            return 0
        lax.fori_loop(0, RPT // 100, _zi, 0)

    plsc.subcore_barrier()

    def issue_loads(i, s):
        base = wid * EPW + i * CK
        pltpu.async_copy(dst_hbm.at[pl.ds(base, CK)], DST[s], SDST[s])
        pltpu.async_copy(ew_hbm.at[pl.ds(base, CK), :], EW[s], SEW[s])

    def wait_loads(s):
        pltpu.make_async_copy(dst_hbm.at[pl.ds(0, CK)], DST[s], SDST[s]).wait()
        pltpu.make_async_copy(ew_hbm.at[pl.ds(0, CK), :], EW[s], SEW[s]).wait()

    def issue_scatters(s):
        pltpu.async_copy(EW[s], ews_s.at[SD[s]], SEWS[s], add=True)
        pltpu.async_copy(one_v, deg_s.at[SD[s]], SONE[s], add=True)

    def wait_scatters(s):
        pltpu.make_async_copy(EW[s], ews_s.at[SD[s]], SEWS[s]).wait()
        pltpu.make_async_copy(one_v, deg_s.at[SD[s]], SONE[s]).wait()

    issue_loads(0, 0)

    def _sub(i, s):
        wait_loads(s)
        for q in range(CK // 16):
            qs = pl.ds(q * 16, 16)
            SD[s][qs] = DST[s][qs]

        @pl.when(i >= 1)
        def _b():
            wait_scatters(1 - s)

        @pl.when(i + 1 < NCHUNK)
        def _a():
            issue_loads(i + 1, 1 - s)

        issue_scatters(s)

    def _body(g, _):
        _sub(2 * g, 0)

        @pl.when(2 * g + 1 < NCHUNK)
        def _odd():
            _sub(2 * g + 1, 1)
        return 0

    lax.fori_loop(0, (NCHUNK + 1) // 2, _body, 0)
    wait_scatters((NCHUNK - 1) % 2)

    plsc.subcore_barrier()

    @pl.when(sid < 10)
    def _out():
        base = sid * RPT
        sl = pl.ds(base, RPT)
        stg = z1_v.at[pl.ds(0, RPT)]
        pltpu.sync_copy(ews_s.at[sl, :], ews_out.at[cid, sl, :])
        pltpu.sync_copy(deg_s.at[sl], stg)
        pltpu.sync_copy(stg, deg_out.at[pl.ds(cid * N + base, RPT)])


_k0 = functools.partial(
    pl.kernel,
    mesh=_mesh,
    compiler_params=pltpu.CompilerParams(needs_layout_passes=False),
    out_type=[
        jax.ShapeDtypeStruct((NC * N,), jnp.float32),
        jax.ShapeDtypeStruct((NC, N, D), jnp.float32),
    ],
    scratch_types=[
        pltpu.VMEM_SHARED((N, D), jnp.float32),
        pltpu.VMEM_SHARED((N,), jnp.float32),
        pltpu.VMEM((CK,), jnp.int32),
        pltpu.VMEM((CK,), jnp.int32),
        pltpu.VMEM((CK,), jnp.int32),
        pltpu.VMEM((CK,), jnp.int32),
        pltpu.VMEM((CK, D), jnp.float32),
        pltpu.VMEM((CK, D), jnp.float32),
        pltpu.VMEM((CK,), jnp.float32),
        pltpu.VMEM((100, D), jnp.float32),
        pltpu.VMEM((1008,), jnp.float32),
    ] + [pltpu.SemaphoreType.DMA] * 8,
)(_k0_body)


CK2 = 16
NCHUNK2 = EPW // CK2


def _k2_body(src_hbm, dst_hbm, ew_hbm, xl_hbm, xr_hbm, att_hbm,
             num_out, den_out,
             num_s, den_s,
             src0, src1, dst0, dst1, sd0, sd1,
             ew0, ew1, xl0, xl1, xr0, xr1, ex0, ex1,
             att_v, att_sp, z_v, z1_v,
             ssrc0, ssrc1, sdst0, sdst1, sew0, sew1,
             sxl0, sxl1, sxr0, sxr1, snum0, snum1, sden0, sden1):
    cid = lax.axis_index("c")
    sid = lax.axis_index("s")
    wid = cid * NS + sid

    SRC = (src0, src1)
    DST = (dst0, dst1)
    SD = (sd0, sd1)
    EW = (ew0, ew1)
    XL = (xl0, xl1)
    XR = (xr0, xr1)
    EX = (ex0, ex1)
    SSRC = (ssrc0, ssrc1)
    SDST = (sdst0, sdst1)
    SEW = (sew0, sew1)
    SXL = (sxl0, sxl1)
    SXR = (sxr0, sxr1)
    SNUM = (snum0, snum1)
    SDEN = (sden0, sden1)

    def _zrow(i, _):
        for c in range(8):
            z_v[i, pl.ds(c * 16, 16)] = jnp.zeros((16,), jnp.float32)
        return 0
    lax.fori_loop(0, 100, _zrow, 0)

    def _z1(i, _):
        z1_v[pl.ds(i * 16, 16)] = jnp.zeros((16,), jnp.float32)
        return 0
    lax.fori_loop(0, 63, _z1, 0)

    pltpu.sync_copy(att_hbm, att_v)

    @pl.when(sid < 10)
    def _init():
        base = sid * RPT
        pltpu.sync_copy(z1_v.at[pl.ds(0, RPT)], den_s.at[pl.ds(base, RPT)])
        def _zi(j, _):
            pltpu.sync_copy(z_v, num_s.at[pl.ds(base + j * 100, 100), :])
            return 0
        lax.fori_loop(0, RPT // 100, _zi, 0)

    plsc.subcore_barrier()

    def issue_loads(i, s):
        base = wid * EPW + i * CK2
        pltpu.async_copy(src_hbm.at[pl.ds(base, CK2)], SRC[s], SSRC[s])
        pltpu.async_copy(dst_hbm.at[pl.ds(base, CK2)], DST[s], SDST[s])
        pltpu.async_copy(ew_hbm.at[pl.ds(base, CK2), :], EW[s], SEW[s])

    def wait_loads(s):
        pltpu.make_async_copy(src_hbm.at[pl.ds(0, CK2)], SRC[s], SSRC[s]).wait()
        pltpu.make_async_copy(dst_hbm.at[pl.ds(0, CK2)], DST[s], SDST[s]).wait()
        pltpu.make_async_copy(ew_hbm.at[pl.ds(0, CK2), :], EW[s], SEW[s]).wait()

    def issue_gathers(s):
        pltpu.async_copy(xl_hbm.at[SRC[s]], XL[s], SXL[s])
        pltpu.async_copy(xr_hbm.at[DST[s]], XR[s], SXR[s])

    def wait_gathers(s):
        pltpu.make_async_copy(xl_hbm.at[SRC[s]], XL[s], SXL[s]).wait()
        pltpu.make_async_copy(xr_hbm.at[DST[s]], XR[s], SXR[s]).wait()

    def issue_scatters(s):
        pltpu.async_copy(XL[s], num_s.at[SD[s]], SNUM[s], add=True)
        pltpu.async_copy(EX[s], den_s.at[SD[s]], SDEN[s], add=True)

    def wait_scatters(s):
        pltpu.make_async_copy(XL[s], num_s.at[SD[s]], SNUM[s]).wait()
        pltpu.make_async_copy(EX[s], den_s.at[SD[s]], SDEN[s]).wait()

    def _stage_att():
        for cc in range(8):
            a16 = att_v[pl.ds(cc * 16, 16)]
            for r in range(16):
                att_sp[cc * 16 + r, pl.ds(0, 16)] = jnp.full(
                    (16,), a16[r], jnp.float32)
    _stage_att()
    lane16 = lax.iota(jnp.int32, 16)

    def compute(s):
        xl_v, xr_v, ew_v = XL[s], XR[s], EW[s]
        SD[s][pl.ds(0, 16)] = DST[s][pl.ds(0, 16)]
        def _cgrp(g, acc):
            for k in range(16):
                c = g * 16 + k
                cc = jnp.full((16,), c, jnp.int32)
                v = (plsc.load_gather(xl_v, [lane16, cc])
                     + plsc.load_gather(xr_v, [lane16, cc])
                     + plsc.load_gather(ew_v, [lane16, cc]))
                v = jnp.maximum(v, 0.2 * v)
                acc = acc + att_sp[c, pl.ds(0, 16)] * v
            return acc
        acc = lax.fori_loop(0, 8, _cgrp, jnp.zeros((16,), jnp.float32))
        ex16 = jnp.exp(acc)
        EX[s][pl.ds(0, 16)] = ex16
        for jj in range(16):
            bc = jnp.full((16,), ex16[jj], jnp.float32)
            for c in range(8):
                cs = pl.ds(c * 16, 16)
                xl_v[jj, cs] = xl_v[jj, cs] * bc

    issue_loads(0, 0)
    wait_loads(0)
    issue_gathers(0)
    issue_loads(1, 1)

    def _sub(i, s):
        wait_gathers(s)
        compute(s)

        @pl.when(i + 1 < NCHUNK2)
        def _a():
            wait_loads(1 - s)

        @pl.when(i >= 1)
        def _b():
            wait_scatters(1 - s)

        issue_scatters(s)

        @pl.when(i + 1 < NCHUNK2)
        def _c():
            issue_gathers(1 - s)

        @pl.when(i + 2 < NCHUNK2)
        def _d():
            issue_loads(i + 2, s)

    def _body(g, _):
        _sub(2 * g, 0)

        @pl.when(2 * g + 1 < NCHUNK2)
        def _odd():
            _sub(2 * g + 1, 1)
        return 0

    lax.fori_loop(0, (NCHUNK2 + 1) // 2, _body, 0)
    wait_scatters((NCHUNK2 - 1) % 2)

    plsc.subcore_barrier()

    @pl.when(sid < 10)
    def _out():
        base = sid * RPT
        sl = pl.ds(base, RPT)
        fsl = pl.ds(cid * N + base, RPT)
        stg = z1_v.at[pl.ds(0, RPT)]
        pltpu.sync_copy(num_s.at[sl, :], num_out.at[cid, sl, :])
        pltpu.sync_copy(den_s.at[sl], stg)
        pltpu.sync_copy(stg, den_out.at[fsl])


_k2 = functools.partial(
    pl.kernel,
    mesh=_mesh,
    compiler_params=pltpu.CompilerParams(needs_layout_passes=False),
    out_type=[
        jax.ShapeDtypeStruct((NC, N, D), jnp.float32),
        jax.ShapeDtypeStruct((NC * N,), jnp.float32),
    ],
    scratch_types=[
        pltpu.VMEM_SHARED((N, D), jnp.float32),
        pltpu.VMEM_SHARED((N,), jnp.float32),
        pltpu.VMEM((CK2,), jnp.int32),
        pltpu.VMEM((CK2,), jnp.int32),
        pltpu.VMEM((CK2,), jnp.int32),
        pltpu.VMEM((CK2,), jnp.int32),
        pltpu.VMEM((CK2,), jnp.int32),
        pltpu.VMEM((CK2,), jnp.int32),
        pltpu.VMEM((CK2, D), jnp.float32),
        pltpu.VMEM((CK2, D), jnp.float32),
        pltpu.VMEM((CK2, D), jnp.float32),
        pltpu.VMEM((CK2, D), jnp.float32),
        pltpu.VMEM((CK2, D), jnp.float32),
        pltpu.VMEM((CK2, D), jnp.float32),
        pltpu.VMEM((CK2,), jnp.float32),
        pltpu.VMEM((CK2,), jnp.float32),
        pltpu.VMEM((D,), jnp.float32),
        pltpu.VMEM((D, 16), jnp.float32),
        pltpu.VMEM((100, D), jnp.float32),
        pltpu.VMEM((1008,), jnp.float32),
    ] + [pltpu.SemaphoreType.DMA] * 14,
)(_k2_body)


def _k3_body(xl_ref, xr_ref, num0_ref, num1_ref, den0_ref, den1_ref,
             deg0_ref, deg1_ref, ews0_ref, ews1_ref,
             att_ref, b1_ref, Wmu_ref, Wls_ref,
             xw_ref, dis_ref):
    xl = xl_ref[...]
    deg = deg0_ref[...] + deg1_ref[...]
    em = (ews0_ref[...] + ews1_ref[...]) / jnp.maximum(deg, 1.0)
    e = xl + xr_ref[...] + em
    e = jnp.maximum(e, 0.2 * e)
    alpha = jnp.dot(e, att_ref[...], preferred_element_type=jnp.float32)
    ex = jnp.exp(alpha)
    den = den0_ref[...] + den1_ref[...] + ex
    num = num0_ref[...] + num1_ref[...] + ex * xl
    h = jnp.maximum(num / den + b1_ref[...], 0.0)
    xw_ref[:, 0:64] = jnp.dot(h, Wmu_ref[...], preferred_element_type=jnp.float32)
    xw_ref[:, 64:128] = jnp.dot(h, Wls_ref[...], preferred_element_type=jnp.float32)
    dis_ref[...] = lax.rsqrt(deg + 1.0)


def _k3(xl, xr, num, den, deg, ews, att, b1, Wmu, Wls):
    B = 1000
    grid = (N // B,)
    r2 = lambda a: a.reshape(N, 1)
    out = pl.pallas_call(
        _k3_body,
        grid=grid,
        in_specs=[
            pl.BlockSpec((B, D), lambda i: (i, 0)),
            pl.BlockSpec((B, D), lambda i: (i, 0)),
            pl.BlockSpec((B, D), lambda i: (i, 0)),
            pl.BlockSpec((B, D), lambda i: (i, 0)),
            pl.BlockSpec((B, 1), lambda i: (i, 0)),
            pl.BlockSpec((B, 1), lambda i: (i, 0)),
            pl.BlockSpec((B, 1), lambda i: (i, 0)),
            pl.BlockSpec((B, 1), lambda i: (i, 0)),
            pl.BlockSpec((B, D), lambda i: (i, 0)),
            pl.BlockSpec((B, D), lambda i: (i, 0)),
            pl.BlockSpec((D, 1), lambda i: (0, 0)),
            pl.BlockSpec((1, D), lambda i: (0, 0)),
            pl.BlockSpec((D, 64), lambda i: (0, 0)),
            pl.BlockSpec((D, 64), lambda i: (0, 0)),
        ],
        out_specs=[
            pl.BlockSpec((B, D), lambda i: (i, 0)),
            pl.BlockSpec((B, 1), lambda i: (i, 0)),
        ],
        out_shape=[
            jax.ShapeDtypeStruct((N, D), jnp.float32),
            jax.ShapeDtypeStruct((N, 1), jnp.float32),
        ],
    )(xl, xr, num[0], num[1], r2(den[:N]), r2(den[N:]), r2(deg[:N]), r2(deg[N:]),
      ews[0], ews[1], att.reshape(D, 1), b1.reshape(1, D), Wmu, Wls)
    return out


def _k4_body(src_hbm, dst_hbm, xw_hbm, dis_hbm,
             acc_out,
             acc_s,
             src0, src1, dst0, dst1, sd0, sd1, xw0, xw1, dis_v, z_v,
             ssrc0, ssrc1, sdst0, sdst1, sxw0, sxw1, ssc0, ssc1):
    cid = lax.axis_index("c")
    sid = lax.axis_index("s")
    wid = cid * NS + sid

    SRC = (src0, src1)
    DST = (dst0, dst1)
    SD = (sd0, sd1)
    XW = (xw0, xw1)
    SSRC = (ssrc0, ssrc1)
    SDST = (sdst0, sdst1)
    SXW = (sxw0, sxw1)
    SSC = (ssc0, ssc1)

    def _zrow(i, _):
        for c in range(8):
            z_v[i, pl.ds(c * 16, 16)] = jnp.zeros((16,), jnp.float32)
        return 0
    lax.fori_loop(0, 100, _zrow, 0)

    pltpu.sync_copy(dis_hbm, dis_v)

    @pl.when(sid < 10)
    def _init():
        base = sid * RPT
        def _zi(j, _):
            pltpu.sync_copy(z_v, acc_s.at[pl.ds(base + j * 100, 100), :])
            return 0
        lax.fori_loop(0, RPT // 100, _zi, 0)

    plsc.subcore_barrier()

    def issue_loads(i, s):
        base = wid * EPW + i * CK
        pltpu.async_copy(src_hbm.at[pl.ds(base, CK)], SRC[s], SSRC[s])
        pltpu.async_copy(dst_hbm.at[pl.ds(base, CK)], DST[s], SDST[s])

    def wait_loads(s):
        pltpu.make_async_copy(src_hbm.at[pl.ds(0, CK)], SRC[s], SSRC[s]).wait()
        pltpu.make_async_copy(dst_hbm.at[pl.ds(0, CK)], DST[s], SDST[s]).wait()

    def issue_gather(s):
        pltpu.async_copy(xw_hbm.at[SRC[s]], XW[s], SXW[s])

    def wait_gather(s):
        pltpu.make_async_copy(xw_hbm.at[SRC[s]], XW[s], SXW[s]).wait()

    def issue_scatter(s):
        pltpu.async_copy(XW[s], acc_s.at[SD[s]], SSC[s], add=True)

    def wait_scatter(s):
        pltpu.make_async_copy(XW[s], acc_s.at[SD[s]], SSC[s]).wait()

    def compute(s):
        xw_v = XW[s]
        for q in range(CK // 16):
            qs = pl.ds(q * 16, 16)
            SD[s][qs] = DST[s][qs]

        def _group(g, _):
            gs = pl.ds(g * 16, 16)
            gl = plsc.load_gather(dis_v, [SRC[s][gs]])
            gr = plsc.load_gather(dis_v, [DST[s][gs]])
            nrm16 = gl * gr
            for jj in range(16):
                bc = jnp.full((16,), nrm16[jj], jnp.float32)
                for c in range(8):
                    cs = pl.ds(c * 16, 16)
                    xw_v[g * 16 + jj, cs] = xw_v[g * 16 + jj, cs] * bc
            return 0
        lax.fori_loop(0, CK // 16, _group, 0)

    issue_loads(0, 0)
    wait_loads(0)
    issue_gather(0)
    issue_loads(1, 1)

    def _sub(i, s):
        wait_gather(s)
        compute(s)

        @pl.when(i + 1 < NCHUNK)
        def _a():
            wait_loads(1 - s)

        @pl.when(i >= 1)
        def _b():
            wait_scatter(1 - s)

        issue_scatter(s)

        @pl.when(i + 1 < NCHUNK)
        def _c():
            issue_gather(1 - s)

        @pl.when(i + 2 < NCHUNK)
        def _d():
            issue_loads(i + 2, s)

    def _body(g, _):
        _sub(2 * g, 0)

        @pl.when(2 * g + 1 < NCHUNK)
        def _odd():
            _sub(2 * g + 1, 1)
        return 0

    lax.fori_loop(0, (NCHUNK + 1) // 2, _body, 0)
    wait_scatter((NCHUNK - 1) % 2)

    plsc.subcore_barrier()

    @pl.when(sid < 10)
    def _out():
        sl = pl.ds(sid * RPT, RPT)
        pltpu.sync_copy(acc_s.at[sl, :], acc_out.at[cid, sl, :])


_k4 = functools.partial(
    pl.kernel,
    mesh=_mesh,
    compiler_params=pltpu.CompilerParams(needs_layout_passes=False),
    out_type=[
        jax.ShapeDtypeStruct((NC, N, D), jnp.float32),
    ],
    scratch_types=[
        pltpu.VMEM_SHARED((N, D), jnp.float32),
        pltpu.VMEM((CK,), jnp.int32),
        pltpu.VMEM((CK,), jnp.int32),
        pltpu.VMEM((CK,), jnp.int32),
        pltpu.VMEM((CK,), jnp.int32),
        pltpu.VMEM((CK,), jnp.int32),
        pltpu.VMEM((CK,), jnp.int32),
        pltpu.VMEM((CK, D), jnp.float32),
        pltpu.VMEM((CK, D), jnp.float32),
        pltpu.VMEM((N,), jnp.float32),
        pltpu.VMEM((100, D), jnp.float32),
    ] + [pltpu.SemaphoreType.DMA] * 8,
)(_k4_body)


def _k5_body(a0_ref, a1_ref, xw_ref, dis_ref, bmu_ref, bls_ref,
             mu_ref, ls_ref):
    d2 = dis_ref[...] * dis_ref[...]
    tot = a0_ref[...] + a1_ref[...] + d2 * xw_ref[...]
    mu_ref[...] = tot[:, 0:64] + bmu_ref[...]
    ls_ref[...] = tot[:, 64:128] + bls_ref[...]


def _k5(acc, xw, dis, bmu, bls):
    B = 1000
    return pl.pallas_call(
        _k5_body,
        grid=(N // B,),
        in_specs=[
            pl.BlockSpec((B, D), lambda i: (i, 0)),
            pl.BlockSpec((B, D), lambda i: (i, 0)),
            pl.BlockSpec((B, D), lambda i: (i, 0)),
            pl.BlockSpec((B, 1), lambda i: (i, 0)),
            pl.BlockSpec((1, 64), lambda i: (0, 0)),
            pl.BlockSpec((1, 64), lambda i: (0, 0)),
        ],
        out_specs=[
            pl.BlockSpec((B, 64), lambda i: (i, 0)),
            pl.BlockSpec((B, 64), lambda i: (i, 0)),
        ],
        out_shape=[
            jax.ShapeDtypeStruct((N, 64), jnp.float32),
            jax.ShapeDtypeStruct((N, 64), jnp.float32),
        ],
    )(acc[0], acc[1], xw, dis, bmu.reshape(1, 64), bls.reshape(1, 64))


def kernel(x, edge_index, edge_attr, Wl, bl, Wr, br, We, att, b1,
           Wmu, bmu, Wls, bls):
    src = edge_index[0]
    dst = edge_index[1]

    xl = _mm(x, Wl, bl, 1000)
    xr = _mm(x, Wr, br, 1000)
    ew = _mm(edge_attr, We, jnp.zeros((D,), jnp.float32), 4000)

    deg, ews = _k0(dst, ew)
    num, den = _k2(src, dst, ew, xl, xr, att)

    xw, dis = _k3(xl, xr, num, den, deg, ews, att, b1, Wmu, Wls)

    (acc,) = _k4(src, dst, xw, dis.reshape(N))

    mu, ls = _k5(acc, xw, dis, bmu, bls)
    return (mu, ls)


# speedup vs baseline: 2.0237x; 2.0237x over previous
import functools

import jax
import jax.numpy as jnp
from jax import lax
from jax.experimental import pallas as pl
from jax.experimental.pallas import tpu as pltpu
from jax.experimental.pallas import tpu_sc as plsc

N = 10000
E = 320000
D = 128
DE = 16

NC = 2
NS = 16
NW = NC * NS
EPW = E // NW
CK = 80
NCHUNK = EPW // CK
RPT = N // 10

_mesh = plsc.VectorSubcoreMesh(core_axis_name="c", subcore_axis_name="s")


def _mm_body(x_ref, w_ref, b_ref, o_ref):
    o_ref[...] = jnp.dot(x_ref[...], w_ref[...],
                         preferred_element_type=jnp.float32) + b_ref[...]


def _mm(x, w, b, block_rows):
    n, d_in = x.shape
    d_out = w.shape[1]
    return pl.pallas_call(
        _mm_body,
        grid=(n // block_rows,),
        in_specs=[
            pl.BlockSpec((block_rows, d_in), lambda i: (i, 0)),
            pl.BlockSpec((d_in, d_out), lambda i: (0, 0)),
            pl.BlockSpec((d_out,), lambda i: (0,)),
        ],
        out_specs=pl.BlockSpec((block_rows, d_out), lambda i: (i, 0)),
        out_shape=jax.ShapeDtypeStruct((n, d_out), jnp.float32),
    )(x, w, b)


def _k0_body(dst_hbm, ew_hbm,
             deg_out, ews_out,
             ews_s, deg_s,
             dst0, dst1, sd0, sd1, ew0, ew1, one_v, z_v, z1_v,
             sdst0, sdst1, sew0, sew1, sews0, sews1, sone0, sone1):
    cid = lax.axis_index("c")
    sid = lax.axis_index("s")
    wid = cid * NS + sid

    DST = (dst0, dst1)
    SD = (sd0, sd1)
    EW = (ew0, ew1)
    SDST = (sdst0, sdst1)
    SEW = (sew0, sew1)
    SEWS = (sews0, sews1)
    SONE = (sone0, sone1)

    def _zrow(i, _):
        for c in range(8):
            z_v[i, pl.ds(c * 16, 16)] = jnp.zeros((16,), jnp.float32)
        return 0
    lax.fori_loop(0, 100, _zrow, 0)

    def _z1(i, _):
        z1_v[pl.ds(i * 16, 16)] = jnp.zeros((16,), jnp.float32)
        return 0
    lax.fori_loop(0, 63, _z1, 0)

    def _ones(i, _):
        one_v[pl.ds(i * 16, 16)] = jnp.ones((16,), jnp.float32)
        return 0
    lax.fori_loop(0, CK // 16, _ones, 0)

    @pl.when(sid < 10)
    def _init():
        base = sid * RPT
        pltpu.sync_copy(z1_v.at[pl.ds(0, RPT)], deg_s.at[pl.ds(base, RPT)])
        def _zi(j, _):
            pltpu.sync_copy(z_v, ews_s.at[pl.ds(base + j * 100, 100), :])
            return 0
        lax.fori_loop(0, RPT // 100, _zi, 0)

    plsc.subcore_barrier()

    def issue_loads(i, s):
        base = wid * EPW + i * CK
        pltpu.async_copy(dst_hbm.at[pl.ds(base, CK)], DST[s], SDST[s])
        pltpu.async_copy(ew_hbm.at[pl.ds(base, CK), :], EW[s], SEW[s])

    def wait_loads(s):
        pltpu.make_async_copy(dst_hbm.at[pl.ds(0, CK)], DST[s], SDST[s]).wait()
        pltpu.make_async_copy(ew_hbm.at[pl.ds(0, CK), :], EW[s], SEW[s]).wait()

    def issue_scatters(s):
        pltpu.async_copy(EW[s], ews_s.at[SD[s]], SEWS[s], add=True)
        pltpu.async_copy(one_v, deg_s.at[SD[s]], SONE[s], add=True)

    def wait_scatters(s):
        pltpu.make_async_copy(EW[s], ews_s.at[SD[s]], SEWS[s]).wait()
        pltpu.make_async_copy(one_v, deg_s.at[SD[s]], SONE[s]).wait()

    issue_loads(0, 0)

    def _sub(i, s):
        wait_loads(s)
        for q in range(CK // 16):
            qs = pl.ds(q * 16, 16)
            SD[s][qs] = DST[s][qs]

        @pl.when(i >= 1)
        def _b():
            wait_scatters(1 - s)

        @pl.when(i + 1 < NCHUNK)
        def _a():
            issue_loads(i + 1, 1 - s)

        issue_scatters(s)

    def _body(g, _):
        _sub(2 * g, 0)

        @pl.when(2 * g + 1 < NCHUNK)
        def _odd():
            _sub(2 * g + 1, 1)
        return 0

    lax.fori_loop(0, (NCHUNK + 1) // 2, _body, 0)
    wait_scatters((NCHUNK - 1) % 2)

    plsc.subcore_barrier()

    @pl.when(sid < 10)
    def _out():
        base = sid * RPT
        sl = pl.ds(base, RPT)
        stg = z1_v.at[pl.ds(0, RPT)]
        pltpu.sync_copy(ews_s.at[sl, :], ews_out.at[cid, sl, :])
        pltpu.sync_copy(deg_s.at[sl], stg)
        pltpu.sync_copy(stg, deg_out.at[pl.ds(cid * N + base, RPT)])


_k0 = functools.partial(
    pl.kernel,
    mesh=_mesh,
    compiler_params=pltpu.CompilerParams(needs_layout_passes=False),
    out_type=[
        jax.ShapeDtypeStruct((NC * N,), jnp.float32),
        jax.ShapeDtypeStruct((NC, N, D), jnp.float32),
    ],
    scratch_types=[
        pltpu.VMEM_SHARED((N, D), jnp.float32),
        pltpu.VMEM_SHARED((N,), jnp.float32),
        pltpu.VMEM((CK,), jnp.int32),
        pltpu.VMEM((CK,), jnp.int32),
        pltpu.VMEM((CK,), jnp.int32),
        pltpu.VMEM((CK,), jnp.int32),
        pltpu.VMEM((CK, D), jnp.float32),
        pltpu.VMEM((CK, D), jnp.float32),
        pltpu.VMEM((CK,), jnp.float32),
        pltpu.VMEM((100, D), jnp.float32),
        pltpu.VMEM((1008,), jnp.float32),
    ] + [pltpu.SemaphoreType.DMA] * 8,
)(_k0_body)


CK2 = 16
NCHUNK2 = EPW // CK2


def _k2_body(src_hbm, dst_hbm, ew_hbm, xl_hbm, xr_hbm, att_hbm,
             num_out, den_out,
             num_s, den_s,
             src0, src1, dst0, dst1, sd0, sd1,
             ew0, ew1, xl0, xl1, xr0, xr1, ex0, ex1,
             att_v, z_v, z1_v,
             ssrc0, ssrc1, sdst0, sdst1, sew0, sew1,
             sxl0, sxl1, sxr0, sxr1, snum0, snum1, sden0, sden1):
    cid = lax.axis_index("c")
    sid = lax.axis_index("s")
    wid = cid * NS + sid

    SRC = (src0, src1)
    DST = (dst0, dst1)
    SD = (sd0, sd1)
    EW = (ew0, ew1)
    XL = (xl0, xl1)
    XR = (xr0, xr1)
    EX = (ex0, ex1)
    SSRC = (ssrc0, ssrc1)
    SDST = (sdst0, sdst1)
    SEW = (sew0, sew1)
    SXL = (sxl0, sxl1)
    SXR = (sxr0, sxr1)
    SNUM = (snum0, snum1)
    SDEN = (sden0, sden1)

    def _zrow(i, _):
        for c in range(8):
            z_v[i, pl.ds(c * 16, 16)] = jnp.zeros((16,), jnp.float32)
        return 0
    lax.fori_loop(0, 100, _zrow, 0)

    def _z1(i, _):
        z1_v[pl.ds(i * 16, 16)] = jnp.zeros((16,), jnp.float32)
        return 0
    lax.fori_loop(0, 63, _z1, 0)

    pltpu.sync_copy(att_hbm, att_v)

    @pl.when(sid < 10)
    def _init():
        base = sid * RPT
        pltpu.sync_copy(z1_v.at[pl.ds(0, RPT)], den_s.at[pl.ds(base, RPT)])
        def _zi(j, _):
            pltpu.sync_copy(z_v, num_s.at[pl.ds(base + j * 100, 100), :])
            return 0
        lax.fori_loop(0, RPT // 100, _zi, 0)

    plsc.subcore_barrier()

    def issue_loads(i, s):
        base = wid * EPW + i * CK2
        pltpu.async_copy(src_hbm.at[pl.ds(base, CK2)], SRC[s], SSRC[s])
        pltpu.async_copy(dst_hbm.at[pl.ds(base, CK2)], DST[s], SDST[s])
        pltpu.async_copy(ew_hbm.at[pl.ds(base, CK2), :], EW[s], SEW[s])

    def wait_loads(s):
        pltpu.make_async_copy(src_hbm.at[pl.ds(0, CK2)], SRC[s], SSRC[s]).wait()
        pltpu.make_async_copy(dst_hbm.at[pl.ds(0, CK2)], DST[s], SDST[s]).wait()
        pltpu.make_async_copy(ew_hbm.at[pl.ds(0, CK2), :], EW[s], SEW[s]).wait()

    def issue_gathers(s):
        pltpu.async_copy(xl_hbm.at[SRC[s]], XL[s], SXL[s])
        pltpu.async_copy(xr_hbm.at[DST[s]], XR[s], SXR[s])

    def wait_gathers(s):
        pltpu.make_async_copy(xl_hbm.at[SRC[s]], XL[s], SXL[s]).wait()
        pltpu.make_async_copy(xr_hbm.at[DST[s]], XR[s], SXR[s]).wait()

    def issue_scatters(s):
        pltpu.async_copy(XL[s], num_s.at[SD[s]], SNUM[s], add=True)
        pltpu.async_copy(EX[s], den_s.at[SD[s]], SDEN[s], add=True)

    def wait_scatters(s):
        pltpu.make_async_copy(XL[s], num_s.at[SD[s]], SNUM[s]).wait()
        pltpu.make_async_copy(EX[s], den_s.at[SD[s]], SDEN[s]).wait()

    def compute(s):
        xl_v, xr_v, ew_v = XL[s], XR[s], EW[s]
        SD[s][pl.ds(0, 16)] = DST[s][pl.ds(0, 16)]
        lane = lax.iota(jnp.int32, 16)
        av = jnp.zeros((16,), jnp.float32)
        for j in range(CK2):
            acc = jnp.zeros((16,), jnp.float32)
            for c in range(8):
                sl = pl.ds(c * 16, 16)
                v = xl_v[j, sl] + xr_v[j, sl] + ew_v[j, sl]
                v = jnp.maximum(v, 0.2 * v)
                acc = acc + att_v[sl] * v
            a = jnp.sum(acc)
            av = jnp.where(lane == j, jnp.full((16,), a, jnp.float32), av)
        ex16 = jnp.exp(av)
        EX[s][pl.ds(0, 16)] = ex16
        for jj in range(16):
            bc = jnp.full((16,), ex16[jj], jnp.float32)
            for c in range(8):
                cs = pl.ds(c * 16, 16)
                xl_v[jj, cs] = xl_v[jj, cs] * bc

    issue_loads(0, 0)
    wait_loads(0)
    issue_gathers(0)
    issue_loads(1, 1)

    def _sub(i, s):
        wait_gathers(s)
        compute(s)

        @pl.when(i + 1 < NCHUNK2)
        def _a():
            wait_loads(1 - s)

        @pl.when(i >= 1)
        def _b():
            wait_scatters(1 - s)

        issue_scatters(s)

        @pl.when(i + 1 < NCHUNK2)
        def _c():
            issue_gathers(1 - s)

        @pl.when(i + 2 < NCHUNK2)
        def _d():
            issue_loads(i + 2, s)

    def _body(g, _):
        _sub(2 * g, 0)

        @pl.when(2 * g + 1 < NCHUNK2)
        def _odd():
            _sub(2 * g + 1, 1)
        return 0

    lax.fori_loop(0, (NCHUNK2 + 1) // 2, _body, 0)
    wait_scatters((NCHUNK2 - 1) % 2)

    plsc.subcore_barrier()

    @pl.when(sid < 10)
    def _out():
        base = sid * RPT
        sl = pl.ds(base, RPT)
        fsl = pl.ds(cid * N + base, RPT)
        stg = z1_v.at[pl.ds(0, RPT)]
        pltpu.sync_copy(num_s.at[sl, :], num_out.at[cid, sl, :])
        pltpu.sync_copy(den_s.at[sl], stg)
        pltpu.sync_copy(stg, den_out.at[fsl])


_k2 = functools.partial(
    pl.kernel,
    mesh=_mesh,
    compiler_params=pltpu.CompilerParams(needs_layout_passes=False),
    out_type=[
        jax.ShapeDtypeStruct((NC, N, D), jnp.float32),
        jax.ShapeDtypeStruct((NC * N,), jnp.float32),
    ],
    scratch_types=[
        pltpu.VMEM_SHARED((N, D), jnp.float32),
        pltpu.VMEM_SHARED((N,), jnp.float32),
        pltpu.VMEM((CK2,), jnp.int32),
        pltpu.VMEM((CK2,), jnp.int32),
        pltpu.VMEM((CK2,), jnp.int32),
        pltpu.VMEM((CK2,), jnp.int32),
        pltpu.VMEM((CK2,), jnp.int32),
        pltpu.VMEM((CK2,), jnp.int32),
        pltpu.VMEM((CK2, D), jnp.float32),
        pltpu.VMEM((CK2, D), jnp.float32),
        pltpu.VMEM((CK2, D), jnp.float32),
        pltpu.VMEM((CK2, D), jnp.float32),
        pltpu.VMEM((CK2, D), jnp.float32),
        pltpu.VMEM((CK2, D), jnp.float32),
        pltpu.VMEM((CK2,), jnp.float32),
        pltpu.VMEM((CK2,), jnp.float32),
        pltpu.VMEM((D,), jnp.float32),
        pltpu.VMEM((100, D), jnp.float32),
        pltpu.VMEM((1008,), jnp.float32),
    ] + [pltpu.SemaphoreType.DMA] * 14,
)(_k2_body)


def _k3_body(xl_ref, xr_ref, num0_ref, num1_ref, den0_ref, den1_ref,
             deg0_ref, deg1_ref, ews0_ref, ews1_ref,
             att_ref, b1_ref, Wmu_ref, Wls_ref,
             xw_ref, dis_ref):
    xl = xl_ref[...]
    deg = deg0_ref[...] + deg1_ref[...]
    em = (ews0_ref[...] + ews1_ref[...]) / jnp.maximum(deg, 1.0)
    e = xl + xr_ref[...] + em
    e = jnp.maximum(e, 0.2 * e)
    alpha = jnp.dot(e, att_ref[...], preferred_element_type=jnp.float32)
    ex = jnp.exp(alpha)
    den = den0_ref[...] + den1_ref[...] + ex
    num = num0_ref[...] + num1_ref[...] + ex * xl
    h = jnp.maximum(num / den + b1_ref[...], 0.0)
    xw_ref[:, 0:64] = jnp.dot(h, Wmu_ref[...], preferred_element_type=jnp.float32)
    xw_ref[:, 64:128] = jnp.dot(h, Wls_ref[...], preferred_element_type=jnp.float32)
    dis_ref[...] = lax.rsqrt(deg + 1.0)


def _k3(xl, xr, num, den, deg, ews, att, b1, Wmu, Wls):
    B = 1000
    grid = (N // B,)
    r2 = lambda a: a.reshape(N, 1)
    out = pl.pallas_call(
        _k3_body,
        grid=grid,
        in_specs=[
            pl.BlockSpec((B, D), lambda i: (i, 0)),
            pl.BlockSpec((B, D), lambda i: (i, 0)),
            pl.BlockSpec((B, D), lambda i: (i, 0)),
            pl.BlockSpec((B, D), lambda i: (i, 0)),
            pl.BlockSpec((B, 1), lambda i: (i, 0)),
            pl.BlockSpec((B, 1), lambda i: (i, 0)),
            pl.BlockSpec((B, 1), lambda i: (i, 0)),
            pl.BlockSpec((B, 1), lambda i: (i, 0)),
            pl.BlockSpec((B, D), lambda i: (i, 0)),
            pl.BlockSpec((B, D), lambda i: (i, 0)),
            pl.BlockSpec((D, 1), lambda i: (0, 0)),
            pl.BlockSpec((1, D), lambda i: (0, 0)),
            pl.BlockSpec((D, 64), lambda i: (0, 0)),
            pl.BlockSpec((D, 64), lambda i: (0, 0)),
        ],
        out_specs=[
            pl.BlockSpec((B, D), lambda i: (i, 0)),
            pl.BlockSpec((B, 1), lambda i: (i, 0)),
        ],
        out_shape=[
            jax.ShapeDtypeStruct((N, D), jnp.float32),
            jax.ShapeDtypeStruct((N, 1), jnp.float32),
        ],
    )(xl, xr, num[0], num[1], r2(den[:N]), r2(den[N:]), r2(deg[:N]), r2(deg[N:]),
      ews[0], ews[1], att.reshape(D, 1), b1.reshape(1, D), Wmu, Wls)
    return out


def _k4_body(src_hbm, dst_hbm, xw_hbm, dis_hbm,
             acc_out,
             acc_s,
             src0, src1, dst0, dst1, sd0, sd1, xw0, xw1, dis_v, z_v,
             ssrc0, ssrc1, sdst0, sdst1, sxw0, sxw1, ssc0, ssc1):
    cid = lax.axis_index("c")
    sid = lax.axis_index("s")
    wid = cid * NS + sid

    SRC = (src0, src1)
    DST = (dst0, dst1)
    SD = (sd0, sd1)
    XW = (xw0, xw1)
    SSRC = (ssrc0, ssrc1)
    SDST = (sdst0, sdst1)
    SXW = (sxw0, sxw1)
    SSC = (ssc0, ssc1)

    def _zrow(i, _):
        for c in range(8):
            z_v[i, pl.ds(c * 16, 16)] = jnp.zeros((16,), jnp.float32)
        return 0
    lax.fori_loop(0, 100, _zrow, 0)

    pltpu.sync_copy(dis_hbm, dis_v)

    @pl.when(sid < 10)
    def _init():
        base = sid * RPT
        def _zi(j, _):
            pltpu.sync_copy(z_v, acc_s.at[pl.ds(base + j * 100, 100), :])
            return 0
        lax.fori_loop(0, RPT // 100, _zi, 0)

    plsc.subcore_barrier()

    def issue_loads(i, s):
        base = wid * EPW + i * CK
        pltpu.async_copy(src_hbm.at[pl.ds(base, CK)], SRC[s], SSRC[s])
        pltpu.async_copy(dst_hbm.at[pl.ds(base, CK)], DST[s], SDST[s])

    def wait_loads(s):
        pltpu.make_async_copy(src_hbm.at[pl.ds(0, CK)], SRC[s], SSRC[s]).wait()
        pltpu.make_async_copy(dst_hbm.at[pl.ds(0, CK)], DST[s], SDST[s]).wait()

    def issue_gather(s):
        pltpu.async_copy(xw_hbm.at[SRC[s]], XW[s], SXW[s])

    def wait_gather(s):
        pltpu.make_async_copy(xw_hbm.at[SRC[s]], XW[s], SXW[s]).wait()

    def issue_scatter(s):
        pltpu.async_copy(XW[s], acc_s.at[SD[s]], SSC[s], add=True)

    def wait_scatter(s):
        pltpu.make_async_copy(XW[s], acc_s.at[SD[s]], SSC[s]).wait()

    def compute(s):
        xw_v = XW[s]
        for q in range(CK // 16):
            qs = pl.ds(q * 16, 16)
            SD[s][qs] = DST[s][qs]

        def _group(g, _):
            gs = pl.ds(g * 16, 16)
            gl = plsc.load_gather(dis_v, [SRC[s][gs]])
            gr = plsc.load_gather(dis_v, [DST[s][gs]])
            nrm16 = gl * gr
            for jj in range(16):
                bc = jnp.full((16,), nrm16[jj], jnp.float32)
                for c in range(8):
                    cs = pl.ds(c * 16, 16)
                    xw_v[g * 16 + jj, cs] = xw_v[g * 16 + jj, cs] * bc
            return 0
        lax.fori_loop(0, CK // 16, _group, 0)

    issue_loads(0, 0)
    wait_loads(0)
    issue_gather(0)
    issue_loads(1, 1)

    def _sub(i, s):
        wait_gather(s)
        compute(s)

        @pl.when(i + 1 < NCHUNK)
        def _a():
            wait_loads(1 - s)

        @pl.when(i >= 1)
        def _b():
            wait_scatter(1 - s)

        issue_scatter(s)

        @pl.when(i + 1 < NCHUNK)
        def _c():
            issue_gather(1 - s)

        @pl.when(i + 2 < NCHUNK)
        def _d():
            issue_loads(i + 2, s)

    def _body(g, _):
        _sub(2 * g, 0)

        @pl.when(2 * g + 1 < NCHUNK)
        def _odd():
            _sub(2 * g + 1, 1)
        return 0

    lax.fori_loop(0, (NCHUNK + 1) // 2, _body, 0)
    wait_scatter((NCHUNK - 1) % 2)

    plsc.subcore_barrier()

    @pl.when(sid < 10)
    def _out():
        sl = pl.ds(sid * RPT, RPT)
        pltpu.sync_copy(acc_s.at[sl, :], acc_out.at[cid, sl, :])


_k4 = functools.partial(
    pl.kernel,
    mesh=_mesh,
    compiler_params=pltpu.CompilerParams(needs_layout_passes=False),
    out_type=[
        jax.ShapeDtypeStruct((NC, N, D), jnp.float32),
    ],
    scratch_types=[
        pltpu.VMEM_SHARED((N, D), jnp.float32),
        pltpu.VMEM((CK,), jnp.int32),
        pltpu.VMEM((CK,), jnp.int32),
        pltpu.VMEM((CK,), jnp.int32),
        pltpu.VMEM((CK,), jnp.int32),
        pltpu.VMEM((CK,), jnp.int32),
        pltpu.VMEM((CK,), jnp.int32),
        pltpu.VMEM((CK, D), jnp.float32),
        pltpu.VMEM((CK, D), jnp.float32),
        pltpu.VMEM((N,), jnp.float32),
        pltpu.VMEM((100, D), jnp.float32),
    ] + [pltpu.SemaphoreType.DMA] * 8,
)(_k4_body)


def _k5_body(a0_ref, a1_ref, xw_ref, dis_ref, bmu_ref, bls_ref,
             mu_ref, ls_ref):
    d2 = dis_ref[...] * dis_ref[...]
    tot = a0_ref[...] + a1_ref[...] + d2 * xw_ref[...]
    mu_ref[...] = tot[:, 0:64] + bmu_ref[...]
    ls_ref[...] = tot[:, 64:128] + bls_ref[...]


def _k5(acc, xw, dis, bmu, bls):
    B = 1000
    return pl.pallas_call(
        _k5_body,
        grid=(N // B,),
        in_specs=[
            pl.BlockSpec((B, D), lambda i: (i, 0)),
            pl.BlockSpec((B, D), lambda i: (i, 0)),
            pl.BlockSpec((B, D), lambda i: (i, 0)),
            pl.BlockSpec((B, 1), lambda i: (i, 0)),
            pl.BlockSpec((1, 64), lambda i: (0, 0)),
            pl.BlockSpec((1, 64), lambda i: (0, 0)),
        ],
        out_specs=[
            pl.BlockSpec((B, 64), lambda i: (i, 0)),
            pl.BlockSpec((B, 64), lambda i: (i, 0)),
        ],
        out_shape=[
            jax.ShapeDtypeStruct((N, 64), jnp.float32),
            jax.ShapeDtypeStruct((N, 64), jnp.float32),
        ],
    )(acc[0], acc[1], xw, dis, bmu.reshape(1, 64), bls.reshape(1, 64))


def kernel(x, edge_index, edge_attr, Wl, bl, Wr, br, We, att, b1,
           Wmu, bmu, Wls, bls):
    src = edge_index[0]
    dst = edge_index[1]

    xl = _mm(x, Wl, bl, 1000)
    xr = _mm(x, Wr, br, 1000)
    ew = _mm(edge_attr, We, jnp.zeros((D,), jnp.float32), 4000)

    deg, ews = _k0(dst, ew)
    num, den = _k2(src, dst, ew, xl, xr, att)

    xw, dis = _k3(xl, xr, num, den, deg, ews, att, b1, Wmu, Wls)

    (acc,) = _k4(src, dst, xw, dis.reshape(N))

    mu, ls = _k5(acc, xw, dis, bmu, bls)
    return (mu, ls)
